# Initial kernel scaffold; baseline (speedup 1.0000x reference)
import jax
import jax.numpy as jnp
from jax.experimental import pallas as pl


def kernel(x):
    raise NotImplementedError("write your pallas kernel here")


# speedup vs baseline: 1.2627x; 1.2627x over previous
import functools

import jax
import jax.numpy as jnp
from jax import lax
from jax.experimental import pallas as pl
from jax.experimental.pallas import tpu as pltpu

_BATCH = 16384
_CNT = 8192
_DIM = 128
_NBINS = 512
_MAX_VAL = 2.0
_MIN_VAL = -2.0
_BIG = 99999
_IDXW = 16384

_TILE = 512
_NT = _CNT // _TILE


def _dmat_minkeys_kernel(a_ref, p_ref, rowk_ref, colk_ref, posd_ref,
                         posb_ref):
    i = pl.program_id(0)
    j = pl.program_id(1)

    a = a_ref[...]
    p = p_ref[...]
    a_s = jnp.where(a > 0, 1.0, -1.0).astype(jnp.bfloat16)
    p_s = jnp.where(p > 0, 1.0, -1.0).astype(jnp.bfloat16)
    dot = lax.dot_general(a_s, p_s, (((1,), (1,)), ((), ())),
                          preferred_element_type=jnp.float32)
    h = ((_DIM - dot) * 0.5).astype(jnp.int32)

    rl = lax.broadcasted_iota(jnp.int32, (_TILE, _TILE), 0)
    cl = lax.broadcasted_iota(jnp.int32, (_TILE, _TILE), 1)
    r = rl + i * _TILE
    c = cl + j * _TILE
    v = h + jnp.where(r == c, _BIG, 0)
    v = jnp.where(v < 2, _BIG, v)

    keys_row = v * _IDXW + c
    keys_col = v * _IDXW + r

    part_row = jnp.min(keys_row, axis=1).reshape(1, _TILE)
    part_col = jnp.min(keys_col, axis=0).reshape(1, _TILE)

    @pl.when(j == 0)
    def _():
        rowk_ref[pl.ds(i, 1), :] = part_row

    @pl.when(j > 0)
    def _():
        rowk_ref[pl.ds(i, 1), :] = jnp.minimum(rowk_ref[pl.ds(i, 1), :],
                                               part_row)

    @pl.when(i == 0)
    def _():
        colk_ref[pl.ds(j, 1), :] = part_col

    @pl.when(i > 0)
    def _():
        colk_ref[pl.ds(j, 1), :] = jnp.minimum(colk_ref[pl.ds(j, 1), :],
                                               part_col)

    @pl.when(i == j)
    def _():
        posd_ref[pl.ds(i, 1), :] = jnp.sum(a * p, axis=1).reshape(1, _TILE)
        diag = jnp.sum(jnp.where(rl == cl, dot, 0.0), axis=1)
        posb_ref[pl.ds(i, 1), :] = diag.reshape(1, _TILE)


def _run_phase_a(x):
    a0 = x[:_CNT]
    p0 = x[_CNT:]
    out_shapes = (
        jax.ShapeDtypeStruct((_NT, _TILE), jnp.int32),
        jax.ShapeDtypeStruct((_NT, _TILE), jnp.int32),
        jax.ShapeDtypeStruct((_NT, _TILE), jnp.float32),
        jax.ShapeDtypeStruct((_NT, _TILE), jnp.float32),
    )
    full = pl.BlockSpec((_NT, _TILE), lambda i, j: (0, 0))
    rowk, colk, posd, posb = pl.pallas_call(
        _dmat_minkeys_kernel,
        grid=(_NT, _NT),
        in_specs=[
            pl.BlockSpec((_TILE, _DIM), lambda i, j: (i, 0)),
            pl.BlockSpec((_TILE, _DIM), lambda i, j: (j, 0)),
        ],
        out_specs=(full, full, full, full),
        out_shape=out_shapes,
        compiler_params=pltpu.CompilerParams(
            dimension_semantics=("arbitrary", "arbitrary")),
    )(a0, p0)
    return (rowk.reshape(_CNT), colk.reshape(_CNT), posd.reshape(_CNT),
            posb.reshape(_CNT))


def _tail_jnp(x, rowk, colk, pos_dot, pos_bdot):
    min_p = rowk // _IDXW
    min_p_idx = rowk % _IDXW
    min_a = colk // _IDXW
    min_a_idx = colk % _IDXW
    mask = min_a < min_p
    j = jnp.arange(_CNT)
    a_row = jnp.where(mask, j + _CNT, j)
    n_row = jnp.where(mask, min_a_idx, min_p_idx + _CNT)
    av = x[a_row]
    nv = x[n_row]
    neg_dot = jnp.sum(av * nv, axis=1)
    neg_bdot = jnp.sum(jnp.where((av > 0) == (nv > 0), 1.0, -1.0), axis=1)

    pos_dist = (_DIM - pos_dot) / _DIM
    neg_dist = (_DIM - neg_dot) / _DIM
    pos_dist_b = (_DIM - pos_bdot) / _DIM
    neg_dist_b = (_DIM - neg_bdot) / _DIM

    hist_var = pos_dist_b - neg_dist_b
    bin_width = (_MAX_VAL - _MIN_VAL) / (_NBINS - 1)
    lo = jnp.floor((hist_var - _MIN_VAL) / bin_width).astype(jnp.int32)
    hi = jnp.clip(lo + 1, 0, _NBINS - 1)
    alpha = 1.0 - (hist_var - _MIN_VAL -
                   lo.astype(jnp.float32) * bin_width) / bin_width
    hist = jnp.zeros(_NBINS,
                     dtype=jnp.float32).at[lo].add(alpha).at[hi].add(1.0 -
                                                                     alpha)
    hist = hist / (hist.sum() + 1e-06)
    pdf = hist / hist.sum()
    cdf = jnp.cumsum(pdf)
    weight = cdf[lo]
    return -(neg_dist * weight).mean() + (pos_dist * weight).mean()


def kernel(x):
    rowk, colk, posd, posb = _run_phase_a(x)
    return _tail_jnp(x, rowk, colk, posd, posb)


# speedup vs baseline: 1.5342x; 1.2150x over previous
import functools

import jax
import jax.numpy as jnp
from jax import lax
from jax.experimental import pallas as pl
from jax.experimental.pallas import tpu as pltpu
from jax.experimental.pallas import tpu_sc as plsc

_BATCH = 16384
_CNT = 8192
_DIM = 128
_NBINS = 512
_MAX_VAL = 2.0
_MIN_VAL = -2.0
_BIG = 99999
_IDXW = 16384

_TILE = 512
_NT = _CNT // _TILE


def _dmat_minkeys_kernel(a_ref, p_ref, rowk_ref, colk_ref, posd_ref,
                         posb_ref):
    i = pl.program_id(0)
    j = pl.program_id(1)

    a = a_ref[...]
    p = p_ref[...]
    a_s = jnp.where(a > 0, 1.0, -1.0).astype(jnp.bfloat16)
    p_s = jnp.where(p > 0, 1.0, -1.0).astype(jnp.bfloat16)
    dot = lax.dot_general(a_s, p_s, (((1,), (1,)), ((), ())),
                          preferred_element_type=jnp.float32)
    h = ((_DIM - dot) * 0.5).astype(jnp.int32)

    rl = lax.broadcasted_iota(jnp.int32, (_TILE, _TILE), 0)
    cl = lax.broadcasted_iota(jnp.int32, (_TILE, _TILE), 1)
    r = rl + i * _TILE
    c = cl + j * _TILE
    v = h + jnp.where(r == c, _BIG, 0)
    v = jnp.where(v < 2, _BIG, v)

    keys_row = v * _IDXW + c
    keys_col = v * _IDXW + r

    part_row = jnp.min(keys_row, axis=1).reshape(1, _TILE)
    part_col = jnp.min(keys_col, axis=0).reshape(1, _TILE)

    @pl.when(j == 0)
    def _():
        rowk_ref[pl.ds(i, 1), :] = part_row

    @pl.when(j > 0)
    def _():
        rowk_ref[pl.ds(i, 1), :] = jnp.minimum(rowk_ref[pl.ds(i, 1), :],
                                               part_row)

    @pl.when(i == 0)
    def _():
        colk_ref[pl.ds(j, 1), :] = part_col

    @pl.when(i > 0)
    def _():
        colk_ref[pl.ds(j, 1), :] = jnp.minimum(colk_ref[pl.ds(j, 1), :],
                                               part_col)

    @pl.when(i == j)
    def _():
        posd_ref[pl.ds(i, 1), :] = jnp.sum(a * p, axis=1).reshape(1, _TILE)
        diag = jnp.sum(jnp.where(rl == cl, dot, 0.0), axis=1)
        posb_ref[pl.ds(i, 1), :] = diag.reshape(1, _TILE)


def _run_phase_a(x):
    a0 = x[:_CNT]
    p0 = x[_CNT:]
    out_shapes = (
        jax.ShapeDtypeStruct((_NT, _TILE), jnp.int32),
        jax.ShapeDtypeStruct((_NT, _TILE), jnp.int32),
        jax.ShapeDtypeStruct((_NT, _TILE), jnp.float32),
        jax.ShapeDtypeStruct((_NT, _TILE), jnp.float32),
    )
    full = pl.BlockSpec((_NT, _TILE), lambda i, j: (0, 0))
    rowk, colk, posd, posb = pl.pallas_call(
        _dmat_minkeys_kernel,
        grid=(_NT, _NT),
        in_specs=[
            pl.BlockSpec((_TILE, _DIM), lambda i, j: (i, 0)),
            pl.BlockSpec((_TILE, _DIM), lambda i, j: (j, 0)),
        ],
        out_specs=(full, full, full, full),
        out_shape=out_shapes,
        compiler_params=pltpu.CompilerParams(
            dimension_semantics=("arbitrary", "arbitrary")),
    )(a0, p0)
    return rowk.reshape(_CNT), colk.reshape(_CNT), posd, posb


_NWORK = 32
_PPW = _CNT // _NWORK


def _gather_neg_sc_kernel(x_hbm, rowk_hbm, colk_hbm, negrows_hbm,
                          rowk_v, colk_v, nidx_v, nrow_v, sem):
    cid = lax.axis_index("c")
    sid = lax.axis_index("s")
    wid = cid * 16 + sid
    base = wid * _PPW

    pltpu.sync_copy(rowk_hbm.at[pl.ds(base, _PPW)], rowk_v)
    pltpu.sync_copy(colk_hbm.at[pl.ds(base, _PPW)], colk_v)

    for t in range(_PPW // 16):
        rk = rowk_v[pl.ds(t * 16, 16)]
        ck = colk_v[pl.ds(t * 16, 16)]
        minp = rk >> 14
        mpidx = rk & (_IDXW - 1)
        mina = ck >> 14
        maidx = ck & (_IDXW - 1)
        nidx = jnp.where(mina < minp, maidx, mpidx + _CNT)
        nidx_v[t // 8, pl.ds((t % 8) * 16, 16)] = nidx

    cps = [
        pltpu.async_copy(x_hbm.at[nidx_v.at[q]],
                         nrow_v.at[pl.ds(q * 128, 128)], sem)
        for q in range(2)
    ]
    for cp in cps:
        cp.wait()

    pltpu.sync_copy(nrow_v, negrows_hbm.at[pl.ds(base, _PPW)])


def _run_phase_b(x, rowk, colk):
    mesh = plsc.VectorSubcoreMesh(core_axis_name="c", subcore_axis_name="s")
    kfn = functools.partial(
        pl.kernel,
        mesh=mesh,
        out_type=jax.ShapeDtypeStruct((_CNT, _DIM), jnp.float32),
        scratch_types=[
            pltpu.VMEM((_PPW,), jnp.int32),
            pltpu.VMEM((_PPW,), jnp.int32),
            pltpu.VMEM((2, 128), jnp.int32),
            pltpu.VMEM((_PPW, _DIM), jnp.float32),
            pltpu.SemaphoreType.DMA,
        ],
    )(_gather_neg_sc_kernel)
    return kfn(x, rowk, colk)


_CH = 128
_NCHUNK = _CNT // _CH


def _loss_kernel(x_ref, neg_ref, rowk_ref, colk_ref, posd_ref, posb_ref,
                 out_ref, negdist_ref, lo_ref):
    bins = lax.broadcasted_iota(jnp.int32, (1, _NBINS), 1)
    bin_width = (_MAX_VAL - _MIN_VAL) / (_NBINS - 1)

    def hist_body(t, hist):
        s = pl.ds(t * _CH, _CH)
        a0c = x_ref[s, :]
        p0c = x_ref[pl.ds(_CNT + t * _CH, _CH), :]
        negc = neg_ref[s, :]
        minp = rowk_ref[s, :] >> 14
        mina = colk_ref[s, :] >> 14
        mask = mina < minp
        anchor = jnp.where(mask, p0c, a0c)
        neg_dot = jnp.sum(anchor * negc, axis=1, keepdims=True)
        bcnt = jnp.sum(jnp.where((anchor > 0) == (negc > 0), 1.0, 0.0),
                       axis=1, keepdims=True)
        neg_bdot = 2.0 * bcnt - _DIM
        neg_dist = (_DIM - neg_dot) * (1.0 / _DIM)
        neg_b = (_DIM - neg_bdot) * (1.0 / _DIM)
        pos_b = (_DIM - posb_ref[s, :]) * (1.0 / _DIM)
        hist_var = pos_b - neg_b
        lo = jnp.floor((hist_var - _MIN_VAL) / bin_width).astype(jnp.int32)
        alpha = 1.0 - (hist_var - _MIN_VAL -
                       lo.astype(jnp.float32) * bin_width) / bin_width
        hi = jnp.clip(lo + 1, 0, _NBINS - 1)
        negdist_ref[s, :] = neg_dist
        lo_ref[s, :] = lo
        contrib = (jnp.where(bins == lo, alpha, 0.0) +
                   jnp.where(bins == hi, 1.0 - alpha, 0.0))
        return hist + jnp.sum(contrib, axis=0, keepdims=True)

    hist = lax.fori_loop(0, _NCHUNK, hist_body,
                         jnp.zeros((1, _NBINS), jnp.float32))
    pdf = hist / jnp.sum(hist)

    def loss_body(t, carry):
        s_pos, s_neg = carry
        s = pl.ds(t * _CH, _CH)
        lo = lo_ref[s, :]
        w = jnp.sum(jnp.where(bins <= lo, pdf, 0.0), axis=1, keepdims=True)
        pos_dist = (_DIM - posd_ref[s, :]) * (1.0 / _DIM)
        return (s_pos + jnp.sum(pos_dist * w, keepdims=True).reshape(1, 1),
                s_neg + jnp.sum(negdist_ref[s, :] * w,
                                keepdims=True).reshape(1, 1))

    zero11 = jnp.zeros((1, 1), jnp.float32)
    s_pos, s_neg = lax.fori_loop(0, _NCHUNK, loss_body, (zero11, zero11))
    out_ref[...] = (s_pos - s_neg) * (1.0 / _CNT)


def _run_phase_c(x, negrows, rowk, colk, posd, posb):
    def col(v, dt):
        return v.reshape(_CNT, 1).astype(dt)

    out = pl.pallas_call(
        _loss_kernel,
        in_specs=[
            pl.BlockSpec((_BATCH, _DIM), lambda: (0, 0)),
            pl.BlockSpec((_CNT, _DIM), lambda: (0, 0)),
            pl.BlockSpec((_CNT, 1), lambda: (0, 0)),
            pl.BlockSpec((_CNT, 1), lambda: (0, 0)),
            pl.BlockSpec((_CNT, 1), lambda: (0, 0)),
            pl.BlockSpec((_CNT, 1), lambda: (0, 0)),
        ],
        out_specs=pl.BlockSpec((1, 1), lambda: (0, 0)),
        out_shape=jax.ShapeDtypeStruct((1, 1), jnp.float32),
        scratch_shapes=[
            pltpu.VMEM((_CNT, 1), jnp.float32),
            pltpu.VMEM((_CNT, 1), jnp.int32),
        ],
    )(x, negrows, col(rowk, jnp.int32), col(colk, jnp.int32),
      col(posd, jnp.float32), col(posb, jnp.float32))
    return out.reshape(())


def kernel(x):
    rowk, colk, posd, posb = _run_phase_a(x)
    negrows = _run_phase_b(x, rowk, colk)
    return _run_phase_c(x, negrows, rowk, colk, posd, posb)
